# baseline (speedup 1.0000x reference)
import functools

import jax
import jax.numpy as jnp
from jax import lax
from jax.experimental import pallas as pl
from jax.experimental.pallas import tpu as pltpu
from jax.experimental.pallas import tpu_sc as plsc

B = 1024
L = 200
D = 1024
SUB = 2
NTOK = L // SUB
NCG = 2
NBW = 16
CW = D // NCG
BB = 32
NBB = B // BB
TB = 50
LANES = 16
VPT = CW // LANES


def _body(idx_hbm, table_hbm, out_hbm,
          idx_v, rows0, rows1, out0, out1, sg0, sg1, so0, so1):
    wid = lax.axis_index("s") * 2 + lax.axis_index("c")
    gc = wid % NCG
    bw = wid // NCG
    col = pl.multiple_of(gc * CW, CW)
    rows = (rows0, rows1)
    outs = (out0, out1)
    sgs = (sg0, sg1)
    sos = (so0, so1)

    def start_gather(tp, par):
        for j in range(SUB):
            pltpu.async_copy(
                table_hbm.at[idx_v.at[tp, j], pl.ds(col, CW)],
                rows[par].at[pl.ds(j * BB, BB)], sgs[par])

    def wait_gather(par):
        for j in range(SUB):
            pltpu.make_async_copy(
                table_hbm.at[idx_v.at[0, 0], pl.ds(col, CW)],
                rows[par].at[pl.ds(0, BB)], sgs[par]).wait()

    def start_out(t, bb, par):
        b0 = pl.multiple_of(bb * BB, BB)
        pltpu.async_copy(
            outs[par], out_hbm.at[t, pl.ds(b0, BB), pl.ds(col, CW)], sos[par])

    def wait_out(par):
        pltpu.make_async_copy(
            outs[par], out_hbm.at[0, pl.ds(0, BB), pl.ds(col, CW)],
            sos[par]).wait()

    def compute(par):
        rv, ov = rows[par], outs[par]

        @plsc.parallel_loop(0, BB, unroll=2)
        def _(q):
            for j in range(VPT):
                s = pl.ds(j * LANES, LANES)
                ov[q, s] = (rv[q, s] + rv[BB + q, s]) * 0.5

    def blkstep(blk, carry):
        bb = bw * 2 + (blk >> 1)
        h = blk & 1
        pltpu.sync_copy(idx_hbm.at[bb, h], idx_v)
        start_gather(0, 0)

        def pairstep(p, c):
            for half in range(2):
                tp = 2 * p + half
                par = half
                wait_gather(par)
                if half == 0:
                    start_gather(tp + 1, 1 - par)
                else:
                    @pl.when(p < TB // 2 - 1)
                    def _():
                        start_gather(tp + 1, 1 - par)

                @pl.when((blk > 0) | (p > 0))
                def _():
                    wait_out(par)
                compute(par)
                start_out(h * TB + tp, bb, par)
            return c

        lax.fori_loop(0, TB // 2, pairstep, 0)
        return carry

    lax.fori_loop(0, 4, blkstep, 0)
    wait_out(0)
    wait_out(1)


_gather_mean = functools.partial(
    pl.kernel,
    out_type=jax.ShapeDtypeStruct((NTOK, B, D), jnp.float32),
    mesh=plsc.VectorSubcoreMesh(core_axis_name="c", subcore_axis_name="s"),
    scratch_types=[
        pltpu.VMEM((TB, SUB, BB), jnp.int32),
        pltpu.VMEM((SUB * BB, CW), jnp.float32),
        pltpu.VMEM((SUB * BB, CW), jnp.float32),
        pltpu.VMEM((BB, CW), jnp.float32),
        pltpu.VMEM((BB, CW), jnp.float32),
        pltpu.SemaphoreType.DMA,
        pltpu.SemaphoreType.DMA,
        pltpu.SemaphoreType.DMA,
        pltpu.SemaphoreType.DMA,
    ],
)(_body)


def kernel(tokens_ids, seg_ids, emb_table):
    idx = (tokens_ids.reshape(NBB, BB, NTOK, SUB)
           .transpose(0, 2, 3, 1)
           .reshape(NBB, 2, TB, SUB, BB))
    out = _gather_mean(idx, emb_table)
    return out.transpose(1, 0, 2)


# speedup vs baseline: 1.0901x; 1.0901x over previous
import functools

import jax
import jax.numpy as jnp
from jax import lax
from jax.experimental import pallas as pl
from jax.experimental.pallas import tpu as pltpu
from jax.experimental.pallas import tpu_sc as plsc

B = 1024
L = 200
D = 1024
SUB = 2
NTOK = L // SUB
NCG = 4
NBW = 8
CW = D // NCG
BB = 64
NBB = B // BB
TB = 50
LANES = 16
VPT = CW // LANES


def _body(idx_hbm, table_hbm, out_hbm,
          idx_v, rows0, rows1, out0, out1, sg0, sg1, so0, so1):
    wid = lax.axis_index("s") * 2 + lax.axis_index("c")
    gc = wid % NCG
    bw = wid // NCG
    col = pl.multiple_of(gc * CW, CW)
    rows = (rows0, rows1)
    outs = (out0, out1)
    sgs = (sg0, sg1)
    sos = (so0, so1)

    def start_gather(tp, par):
        for j in range(SUB):
            pltpu.async_copy(
                table_hbm.at[idx_v.at[tp, j], pl.ds(col, CW)],
                rows[par].at[pl.ds(j * BB, BB)], sgs[par])

    def wait_gather(par):
        for j in range(SUB):
            pltpu.make_async_copy(
                table_hbm.at[idx_v.at[0, 0], pl.ds(col, CW)],
                rows[par].at[pl.ds(0, BB)], sgs[par]).wait()

    def start_out(t, bb, par):
        b0 = pl.multiple_of(bb * BB, BB)
        pltpu.async_copy(
            outs[par], out_hbm.at[t, pl.ds(b0, BB), pl.ds(col, CW)], sos[par])

    def wait_out(par):
        pltpu.make_async_copy(
            outs[par], out_hbm.at[0, pl.ds(0, BB), pl.ds(col, CW)],
            sos[par]).wait()

    def compute(par):
        rv, ov = rows[par], outs[par]

        @plsc.parallel_loop(0, BB, unroll=2)
        def _(q):
            for j in range(VPT):
                s = pl.ds(j * LANES, LANES)
                ov[q, s] = (rv[q, s] + rv[BB + q, s]) * 0.5

    def blkstep(blk, carry):
        bb = bw * 2 + (blk >> 1)
        h = blk & 1
        pltpu.sync_copy(idx_hbm.at[bb, h], idx_v)
        start_gather(0, 0)

        def pairstep(p, c):
            for half in range(2):
                tp = 2 * p + half
                par = half
                wait_gather(par)
                if half == 0:
                    start_gather(tp + 1, 1 - par)
                else:
                    @pl.when(p < TB // 2 - 1)
                    def _():
                        start_gather(tp + 1, 1 - par)

                @pl.when((blk > 0) | (p > 0))
                def _():
                    wait_out(par)
                compute(par)
                start_out(h * TB + tp, bb, par)
            return c

        lax.fori_loop(0, TB // 2, pairstep, 0)
        return carry

    lax.fori_loop(0, 4, blkstep, 0)
    wait_out(0)
    wait_out(1)


_gather_mean = functools.partial(
    pl.kernel,
    out_type=jax.ShapeDtypeStruct((NTOK, B, D), jnp.float32),
    mesh=plsc.VectorSubcoreMesh(core_axis_name="c", subcore_axis_name="s"),
    scratch_types=[
        pltpu.VMEM((TB, SUB, BB), jnp.int32),
        pltpu.VMEM((SUB * BB, CW), jnp.float32),
        pltpu.VMEM((SUB * BB, CW), jnp.float32),
        pltpu.VMEM((BB, CW), jnp.float32),
        pltpu.VMEM((BB, CW), jnp.float32),
        pltpu.SemaphoreType.DMA,
        pltpu.SemaphoreType.DMA,
        pltpu.SemaphoreType.DMA,
        pltpu.SemaphoreType.DMA,
    ],
)(_body)


def kernel(tokens_ids, seg_ids, emb_table):
    idx = (tokens_ids.reshape(NBB, BB, NTOK, SUB)
           .transpose(0, 2, 3, 1)
           .reshape(NBB, 2, TB, SUB, BB))
    out = _gather_mean(idx, emb_table)
    return out.transpose(1, 0, 2)
